# baseline (speedup 1.0000x reference)
import functools

import jax
import jax.numpy as jnp
from jax import lax
from jax.experimental import pallas as pl
from jax.experimental.pallas import tpu as pltpu
from jax.experimental.pallas import tpu_sc as plsc

_D = 256
_MW = 50


def _build_sc_copy(N, T):
    info = plsc.get_sparse_core_info()
    NC, NS, L = info.num_cores, info.num_subcores, info.num_lanes
    NW = NC * NS
    NPW = N // NW
    RPW = NPW * _MW

    mesh = plsc.VectorSubcoreMesh(core_axis_name="c", subcore_axis_name="s")

    @functools.partial(
        pl.kernel,
        mesh=mesh,
        compiler_params=pltpu.CompilerParams(use_tc_tiling_on_sc=False),
        out_type=jax.ShapeDtypeStruct((N * _MW, _D), jnp.float32),
        scratch_types=[
            pltpu.VMEM((NPW,), jnp.int32),
            pltpu.VMEM((NPW,), jnp.int32),
            pltpu.SemaphoreType.DMA,
        ],
    )
    def sc_copy(starts_hbm, ends_hbm, table_hbm, zeros_hbm, out_hbm,
                starts_v, ends_v, dsem):
        wid = lax.axis_index("s") * NC + lax.axis_index("c")
        nbase = pl.multiple_of(wid * NPW, NPW)
        pltpu.sync_copy(starts_hbm.at[pl.ds(nbase, NPW)], starts_v)
        pltpu.sync_copy(ends_hbm.at[pl.ds(nbase, NPW)], ends_v)

        rbase = wid * RPW

        def mk_branch(k, eff, obase):
            def br():
                pltpu.async_copy(
                    table_hbm.at[pl.ds(eff, k)],
                    out_hbm.at[pl.ds(obase, k)], dsem)
                if k < _MW:
                    pltpu.async_copy(
                        zeros_hbm.at[pl.ds(0, _MW - k)],
                        out_hbm.at[pl.ds(obase + k, _MW - k)], dsem)
            return br

        def group_body(g, _):
            goff = pl.multiple_of(g * L, L)
            sv = starts_v[pl.ds(goff, L)]
            ev = ends_v[pl.ds(goff, L)]
            for i in range(L):
                s = sv[i]
                e = ev[i]
                meta = e < 0
                eff = jnp.where(meta, T - 1, s)
                cnt = jnp.where(meta, 1, e - s + 1)
                obase = rbase + (g * L + i) * _MW
                lax.switch(cnt - 1,
                           [mk_branch(k, eff, obase)
                            for k in range(1, _MW + 1)])
            return 0

        lax.fori_loop(0, NPW // L, group_body, 0)

        myout = out_hbm.at[pl.ds(pl.multiple_of(rbase, RPW), RPW)]
        pltpu.make_async_copy(myout, myout, dsem).wait()

    return sc_copy


def kernel(batched_nodes, batched_bi_lstm_outputs):
    nodes0 = batched_nodes[0]
    H = batched_bi_lstm_outputs[0]
    N = nodes0.shape[0]
    T = H.shape[0]
    starts = nodes0[:, 0]
    ends = nodes0[:, 1]
    zeros = jnp.zeros((_MW, _D), H.dtype)
    out = _build_sc_copy(N, T)(starts, ends, H, zeros)
    return out.reshape(1, N, _MW, _D)


# speedup vs baseline: 13.1178x; 13.1178x over previous
import functools

import jax
import jax.numpy as jnp
from jax import lax
from jax.experimental import pallas as pl
from jax.experimental.pallas import tpu as pltpu
from jax.experimental.pallas import tpu_sc as plsc

_D = 256
_MW = 50


def _build_sc_copy(N, T):
    info = plsc.get_sparse_core_info()
    NC, NS, L = info.num_cores, info.num_subcores, info.num_lanes
    NW = NC * NS
    NPW = N // NW
    RPW = NPW * _MW

    mesh = plsc.VectorSubcoreMesh(core_axis_name="c", subcore_axis_name="s")

    @functools.partial(
        pl.kernel,
        mesh=mesh,
        compiler_params=pltpu.CompilerParams(use_tc_tiling_on_sc=False),
        out_type=jax.ShapeDtypeStruct((N * _MW, _D), jnp.float32),
        scratch_types=[
            pltpu.VMEM((NPW,), jnp.int32),
            pltpu.VMEM((NPW,), jnp.int32),
            pltpu.VMEM_SHARED((T, _D), jnp.float32),
            pltpu.VMEM_SHARED((_MW, _D), jnp.float32),
            pltpu.SemaphoreType.DMA,
        ],
    )
    def sc_copy(starts_hbm, ends_hbm, table_hbm, zeros_hbm, out_hbm,
                starts_v, ends_v, sp_table, sp_zeros, dsem):
        wid = lax.axis_index("s") * NC + lax.axis_index("c")
        sid = lax.axis_index("s")
        nbase = pl.multiple_of(wid * NPW, NPW)
        pltpu.sync_copy(starts_hbm.at[pl.ds(nbase, NPW)], starts_v)
        pltpu.sync_copy(ends_hbm.at[pl.ds(nbase, NPW)], ends_v)

        TROWS = T // NS
        soff = pl.multiple_of(sid * TROWS, TROWS)
        pltpu.sync_copy(table_hbm.at[pl.ds(soff, TROWS)],
                        sp_table.at[pl.ds(soff, TROWS)])

        @pl.when(sid == 0)
        def _():
            pltpu.sync_copy(zeros_hbm, sp_zeros)

        plsc.subcore_barrier()

        rbase = wid * RPW

        def mk_branch(k, eff, obase):
            def br():
                pltpu.async_copy(
                    sp_table.at[pl.ds(eff, k)],
                    out_hbm.at[pl.ds(obase, k)], dsem)
                if k < _MW:
                    pltpu.async_copy(
                        sp_zeros.at[pl.ds(0, _MW - k)],
                        out_hbm.at[pl.ds(obase + k, _MW - k)], dsem)
            return br

        def group_body(g, _):
            goff = pl.multiple_of(g * L, L)
            sv = starts_v[pl.ds(goff, L)]
            ev = ends_v[pl.ds(goff, L)]
            for i in range(L):
                s = sv[i]
                e = ev[i]
                meta = e < 0
                eff = jnp.where(meta, T - 1, s)
                cnt = jnp.where(meta, 1, e - s + 1)
                obase = rbase + (g * L + i) * _MW
                lax.switch(cnt - 1,
                           [mk_branch(k, eff, obase)
                            for k in range(1, _MW + 1)])
            return 0

        lax.fori_loop(0, NPW // L, group_body, 0)

        myout = out_hbm.at[pl.ds(pl.multiple_of(rbase, RPW), RPW)]
        pltpu.make_async_copy(myout, myout, dsem).wait()

    return sc_copy


def kernel(batched_nodes, batched_bi_lstm_outputs):
    nodes0 = batched_nodes[0]
    H = batched_bi_lstm_outputs[0]
    N = nodes0.shape[0]
    T = H.shape[0]
    starts = nodes0[:, 0]
    ends = nodes0[:, 1]
    zeros = jnp.zeros((_MW, _D), H.dtype)
    out = _build_sc_copy(N, T)(starts, ends, H, zeros)
    return out.reshape(1, N, _MW, _D)
